# Initial kernel scaffold; baseline (speedup 1.0000x reference)
import jax
import jax.numpy as jnp
from jax.experimental import pallas as pl


def kernel(subs, mask, embed_table):
    raise NotImplementedError("write your pallas kernel here")


# speedup vs baseline: 3.6117x; 3.6117x over previous
import jax
import jax.numpy as jnp
from jax.experimental import pallas as pl

_B, _N = 4096, 50
_MAX_OUT = 4
_D = 64
_NUM_EMB = 8
_BB = 256


def _emb_kernel(subs_ref, mask_ref, tab_ref, buf_ref, cnt_ref):
    subs = subs_ref[...]
    mf = mask_ref[...].astype(jnp.float32)
    ids = jnp.clip(subs + 1, 0, _NUM_EMB - 1)
    k_iota = jax.lax.broadcasted_iota(jnp.int32, (1, 1, _NUM_EMB), 2)
    oh = (ids[..., None] == k_iota).astype(jnp.float32) * mf[..., None]
    prim = jax.lax.dot_general(
        oh.reshape(_BB * _N, _NUM_EMB), tab_ref[...],
        (((1,), (0,)), ((), ())), preferred_element_type=jnp.float32)
    buf_ref[:, :, 0:_D] = prim.reshape(_BB, _N, _D)
    buf_ref[:, :, _D:] = jnp.zeros((_BB, _N, (_MAX_OUT - 1) * _D), jnp.float32)
    cnt_ref[...] = mf


def kernel(subs, mask, embed_table):
    mask_i = mask.astype(jnp.int32)
    grid = (_B // _BB,)
    buf, cnt = pl.pallas_call(
        _emb_kernel,
        grid=grid,
        in_specs=[
            pl.BlockSpec((_BB, _N), lambda i: (i, 0)),
            pl.BlockSpec((_BB, _N), lambda i: (i, 0)),
            pl.BlockSpec((_NUM_EMB, _D), lambda i: (0, 0)),
        ],
        out_specs=[
            pl.BlockSpec((_BB, _N, _MAX_OUT * _D), lambda i: (i, 0, 0)),
            pl.BlockSpec((_BB, _N), lambda i: (i, 0)),
        ],
        out_shape=[
            jax.ShapeDtypeStruct((_B, _N, _MAX_OUT * _D), jnp.float32),
            jax.ShapeDtypeStruct((_B, _N), jnp.float32),
        ],
    )(subs, mask_i, embed_table)
    return buf.reshape(_B, _N, _MAX_OUT, _D), cnt
